# baseline (speedup 1.0000x reference)
import functools

import jax
import jax.numpy as jnp
from jax import lax
from jax.experimental import pallas as pl
from jax.experimental.pallas import tpu as pltpu
from jax.experimental.pallas import tpu_sc as plsc

N_TOKENS = 32768
DIM = 1024
NPAIR = DIM // 2

NC = 2
NS = 16
NW = NC * NS
L = 16

SC_ROWS = 4096
TC_ROWS = N_TOKENS - SC_ROWS
BM = 512
ROWS_PER_W = SC_ROWS // NW
T = 16
TS = T // 8
NTILES = ROWS_PER_W // T
NCHUNK = NPAIR // L
NBI = 4
NBO = 2


def _body(x_hbm, i0h_hbm, i0l_hbm, i1h_hbm, i1l_hbm, jah_hbm, jal_hbm,
          jbh_hbm, jbl_hbm, c_hbm, s_hbm, out_hbm,
          xi0, xi1, xi2, xi3, ot0, ot1,
          i0hv, i0lv, i1hv, i1lv, jahv, jalv, jbhv, jblv, cv, sv,
          si0, si1, si2, si3, so0, so1):
    wid = lax.axis_index("s") * NC + lax.axis_index("c")
    slab0 = wid * (ROWS_PER_W // 8)
    bufs_in = [xi0, xi1, xi2, xi3]
    sems_in = [si0, si1, si2, si3]
    bufs_out = [ot0, ot1]
    sems_out = [so0, so1]

    pltpu.sync_copy(i0h_hbm, i0hv)
    pltpu.sync_copy(i0l_hbm, i0lv)
    pltpu.sync_copy(i1h_hbm, i1hv)
    pltpu.sync_copy(i1l_hbm, i1lv)
    pltpu.sync_copy(jah_hbm, jahv)
    pltpu.sync_copy(jal_hbm, jalv)
    pltpu.sync_copy(jbh_hbm, jbhv)
    pltpu.sync_copy(jbl_hbm, jblv)
    pltpu.sync_copy(c_hbm, cv)
    pltpu.sync_copy(s_hbm, sv)

    def in_slice(g):
        return x_hbm.at[pl.ds(slab0 + g * TS, TS)]

    def out_slice(g):
        return out_hbm.at[pl.ds(slab0 + g * TS, TS)]

    def compute(xt, ot):
        @plsc.parallel_loop(0, NCHUNK)
        def chunk_body(pc):
            o = pc * L
            i0c = i0hv[pl.ds(o, L)]
            i1c = i1hv[pl.ds(o, L)]
            jac = jahv[pl.ds(o, L)]
            jbc = jbhv[pl.ds(o, L)]
            cc = cv[pl.ds(o, L)]
            sc = sv[pl.ds(o, L)]
            _ = (i0lv, i1lv, jalv, jblv)

            @plsc.parallel_loop(0, T, unroll=8)
            def row_body(r):
                tg = jnp.full((L,), r // 8, dtype=jnp.int32)
                ri = jnp.full((L,), r % 8, dtype=jnp.int32)
                xi = plsc.load_gather(xt, [tg, ri, i0c])
                xj = plsc.load_gather(xt, [tg, ri, i1c])
                plsc.store_scatter(ot, [tg, ri, jac], cc * xi - sc * xj)
                plsc.store_scatter(ot, [tg, ri, jbc], cc * xj + sc * xi)

    for b in range(NBI - 1):
        pltpu.async_copy(in_slice(b), bufs_in[b], sems_in[b])

    def quad_body(q, _):
        for b in range(NBI):
            g = NBI * q + b
            pb = (b + NBI - 1) % NBI

            @pl.when(g + NBI - 1 < NTILES)
            def _(g=g, pb=pb):
                pltpu.async_copy(in_slice(g + NBI - 1), bufs_in[pb],
                                 sems_in[pb])

            pltpu.make_async_copy(in_slice(g), bufs_in[b], sems_in[b]).wait()
            ob = b % NBO

            @pl.when(g >= NBO)
            def _(g=g, ob=ob):
                pltpu.make_async_copy(bufs_out[ob], out_slice(g - NBO),
                                      sems_out[ob]).wait()

            compute(bufs_in[b], bufs_out[ob])
            pltpu.async_copy(bufs_out[ob], out_slice(g), sems_out[ob])
        return 0

    lax.fori_loop(0, NTILES // NBI, quad_body, 0)
    pltpu.make_async_copy(bufs_out[0], out_slice(NTILES - 2), sems_out[0]).wait()
    pltpu.make_async_copy(bufs_out[1], out_slice(NTILES - 1), sems_out[1]).wait()


def _tc_body(x_ref, w_ref, y_ref, o_ref):
    del y_ref
    o_ref[...] = jnp.dot(x_ref[...].astype(jnp.bfloat16), w_ref[...],
                         preferred_element_type=jnp.float32)


@jax.jit
def _run(x, tabs, c, s, w):
    mesh = plsc.VectorSubcoreMesh(
        core_axis_name="c", subcore_axis_name="s", num_cores=NC,
        num_subcores=NS)
    f = pl.kernel(
        _body,
        out_type=jax.ShapeDtypeStruct((N_TOKENS // 8, 8, DIM), jnp.float32),
        mesh=mesh,
        compiler_params=pltpu.CompilerParams(needs_layout_passes=False),
        scratch_types=(
            [pltpu.VMEM((TS, 8, DIM), jnp.float32)] * NBI
            + [pltpu.VMEM((TS, 8, DIM), jnp.float32)] * NBO
            + [pltpu.VMEM((NPAIR,), jnp.int32)] * 8
            + [pltpu.VMEM((NPAIR,), jnp.float32)] * 2
            + [pltpu.SemaphoreType.DMA] * (NBI + NBO)
        ),
    )
    x3 = x.reshape(N_TOKENS // 8, 8, DIM)
    sc_out = f(x3, *tabs, c, s).reshape(N_TOKENS, DIM)
    if TC_ROWS == 0:
        return sc_out

    off = SC_ROWS // BM
    out = pl.pallas_call(
        _tc_body,
        grid=(TC_ROWS // BM,),
        in_specs=[
            pl.BlockSpec((BM, DIM), lambda i: (i + off, 0)),
            pl.BlockSpec((DIM, DIM), lambda i: (0, 0)),
            pl.BlockSpec(memory_space=pl.ANY),
        ],
        out_specs=pl.BlockSpec((BM, DIM), lambda i: (i + off, 0)),
        out_shape=jax.ShapeDtypeStruct((N_TOKENS, DIM), jnp.float32),
        input_output_aliases={2: 0},
    )(x, w, sc_out)
    return out


def kernel(x, thetas, inp_pairs, outp_inds):
    c = jnp.cos(thetas)
    s = jnp.sin(thetas)
    i0 = inp_pairs[:, 0]
    i1 = inp_pairs[:, 1]
    inv = jnp.zeros((DIM,), jnp.int32).at[outp_inds].set(
        jnp.arange(DIM, dtype=jnp.int32))
    ja = inv[:NPAIR]
    jb = inv[NPAIR:]

    p = outp_inds % NPAIR
    lo = outp_inds < NPAIR
    ia = jnp.where(lo, i0[p], i1[p])
    ib = jnp.where(lo, i1[p], i0[p])
    av = c[p]
    bv = jnp.where(lo, -s[p], s[p])
    cols = jnp.arange(DIM, dtype=jnp.int32)
    w = (jnp.zeros((DIM, DIM), jnp.float32)
         .at[ia, cols].set(av)
         .at[ib, cols].set(bv)
         .astype(jnp.bfloat16))
    z = jnp.zeros((NPAIR,), jnp.int32)
    tabs = (i0, z, i1, z, ja, z, jb, z)
    return _run(x, tabs, c, s, w)


# speedup vs baseline: 1.0781x; 1.0781x over previous
import functools

import jax
import jax.numpy as jnp
from jax import lax
from jax.experimental import pallas as pl
from jax.experimental.pallas import tpu as pltpu
from jax.experimental.pallas import tpu_sc as plsc

N_TOKENS = 32768
DIM = 1024
NPAIR = DIM // 2

NC = 2
NS = 16
NW = NC * NS
L = 16

SC_ROWS = 8192
TC_ROWS = N_TOKENS - SC_ROWS
BM = 512
ROWS_PER_W = SC_ROWS // NW
T = 16
TS = T // 8
NTILES = ROWS_PER_W // T
NCHUNK = NPAIR // L
NBI = 4
NBO = 2


def _body(x_hbm, i0_hbm, i1_hbm, ja_hbm, jb_hbm, c_hbm, s_hbm, out_hbm,
          xi0, xi1, xi2, xi3, ot0, ot1,
          i0v, i1v, jav, jbv, cv, sv,
          si0, si1, si2, si3, so0, so1):
    wid = lax.axis_index("s") * NC + lax.axis_index("c")
    slab0 = wid * (ROWS_PER_W // 8)
    bufs_in = [xi0, xi1, xi2, xi3]
    sems_in = [si0, si1, si2, si3]
    bufs_out = [ot0, ot1]
    sems_out = [so0, so1]

    pltpu.sync_copy(i0_hbm, i0v)
    pltpu.sync_copy(i1_hbm, i1v)
    pltpu.sync_copy(ja_hbm, jav)
    pltpu.sync_copy(jb_hbm, jbv)
    pltpu.sync_copy(c_hbm, cv)
    pltpu.sync_copy(s_hbm, sv)

    def in_slice(g):
        return x_hbm.at[pl.ds(slab0 + g * TS, TS)]

    def out_slice(g):
        return out_hbm.at[pl.ds(slab0 + g * TS, TS)]

    def compute(xt, ot):
        @plsc.parallel_loop(0, NCHUNK)
        def chunk_body(pc):
            o = pc * L
            i0c = i0v[pl.ds(o, L)]
            i1c = i1v[pl.ds(o, L)]
            jac = jav[pl.ds(o, L)]
            jbc = jbv[pl.ds(o, L)]
            cc = cv[pl.ds(o, L)]
            sc = sv[pl.ds(o, L)]

            @plsc.parallel_loop(0, T, unroll=8)
            def row_body(r):
                tg = jnp.full((L,), r // 8, dtype=jnp.int32)
                ri = jnp.full((L,), r % 8, dtype=jnp.int32)
                xi = plsc.load_gather(xt, [tg, ri, i0c])
                xj = plsc.load_gather(xt, [tg, ri, i1c])
                plsc.store_scatter(ot, [tg, ri, jac], cc * xi - sc * xj)
                plsc.store_scatter(ot, [tg, ri, jbc], cc * xj + sc * xi)

    for b in range(NBI - 1):
        pltpu.async_copy(in_slice(b), bufs_in[b], sems_in[b])

    def quad_body(q, _):
        for b in range(NBI):
            g = NBI * q + b
            pb = (b + NBI - 1) % NBI

            @pl.when(g + NBI - 1 < NTILES)
            def _(g=g, pb=pb):
                pltpu.async_copy(in_slice(g + NBI - 1), bufs_in[pb],
                                 sems_in[pb])

            pltpu.make_async_copy(in_slice(g), bufs_in[b], sems_in[b]).wait()
            ob = b % NBO

            @pl.when(g >= NBO)
            def _(g=g, ob=ob):
                pltpu.make_async_copy(bufs_out[ob], out_slice(g - NBO),
                                      sems_out[ob]).wait()

            compute(bufs_in[b], bufs_out[ob])
            pltpu.async_copy(bufs_out[ob], out_slice(g), sems_out[ob])
        return 0

    lax.fori_loop(0, NTILES // NBI, quad_body, 0)
    pltpu.make_async_copy(bufs_out[0], out_slice(NTILES - 2), sems_out[0]).wait()
    pltpu.make_async_copy(bufs_out[1], out_slice(NTILES - 1), sems_out[1]).wait()


def _tc_body(x_ref, w_ref, y_ref, o_ref):
    del y_ref
    o_ref[...] = jnp.dot(x_ref[...].astype(jnp.bfloat16), w_ref[...],
                         preferred_element_type=jnp.float32)


@jax.jit
def _run(x, tabs, c, s, w):
    mesh = plsc.VectorSubcoreMesh(
        core_axis_name="c", subcore_axis_name="s", num_cores=NC,
        num_subcores=NS)
    f = pl.kernel(
        _body,
        out_type=jax.ShapeDtypeStruct((N_TOKENS // 8, 8, DIM), jnp.float32),
        mesh=mesh,
        compiler_params=pltpu.CompilerParams(needs_layout_passes=False),
        scratch_types=(
            [pltpu.VMEM((TS, 8, DIM), jnp.float32)] * NBI
            + [pltpu.VMEM((TS, 8, DIM), jnp.float32)] * NBO
            + [pltpu.VMEM((NPAIR,), jnp.int32)] * 4
            + [pltpu.VMEM((NPAIR,), jnp.float32)] * 2
            + [pltpu.SemaphoreType.DMA] * (NBI + NBO)
        ),
    )
    x3 = x.reshape(N_TOKENS // 8, 8, DIM)
    sc_out = f(x3, *tabs, c, s).reshape(N_TOKENS, DIM)
    if TC_ROWS == 0:
        return sc_out

    off = SC_ROWS // BM
    out = pl.pallas_call(
        _tc_body,
        grid=(TC_ROWS // BM,),
        in_specs=[
            pl.BlockSpec((BM, DIM), lambda i: (i + off, 0)),
            pl.BlockSpec((DIM, DIM), lambda i: (0, 0)),
            pl.BlockSpec(memory_space=pl.ANY),
        ],
        out_specs=pl.BlockSpec((BM, DIM), lambda i: (i + off, 0)),
        out_shape=jax.ShapeDtypeStruct((N_TOKENS, DIM), jnp.float32),
        input_output_aliases={2: 0},
    )(x, w, sc_out)
    return out


def kernel(x, thetas, inp_pairs, outp_inds):
    c = jnp.cos(thetas)
    s = jnp.sin(thetas)
    i0 = inp_pairs[:, 0]
    i1 = inp_pairs[:, 1]
    inv = jnp.zeros((DIM,), jnp.int32).at[outp_inds].set(
        jnp.arange(DIM, dtype=jnp.int32))
    ja = inv[:NPAIR]
    jb = inv[NPAIR:]

    p = outp_inds % NPAIR
    lo = outp_inds < NPAIR
    ia = jnp.where(lo, i0[p], i1[p])
    ib = jnp.where(lo, i1[p], i0[p])
    av = c[p]
    bv = jnp.where(lo, -s[p], s[p])
    cols = jnp.arange(DIM, dtype=jnp.int32)
    w = (jnp.zeros((DIM, DIM), jnp.float32)
         .at[ia, cols].set(av)
         .at[ib, cols].set(bv)
         .astype(jnp.bfloat16))
    return _run(x, (i0, i1, ja, jb), c, s, w)
